# Initial kernel scaffold; baseline (speedup 1.0000x reference)
import jax
import jax.numpy as jnp
from jax.experimental import pallas as pl


def kernel(x, W_r, W1, b1, W2, b2):
    raise NotImplementedError("write your pallas kernel here")


# speedup vs baseline: 3.3706x; 3.3706x over previous
import functools

import jax
import jax.numpy as jnp
from jax import lax
from jax.experimental import pallas as pl
from jax.experimental.pallas import tpu as pltpu
from jax.experimental.pallas import tpu_sc as plsc

N_TOK_BLK = 2048
CAP = 128
SIGN = -2147483648
FF_BLK = 512

_sc_info = plsc.get_sparse_core_info()
NC, NS, L = _sc_info.num_cores, _sc_info.num_subcores, _sc_info.num_lanes
NW = NC * NS


def _router_body(wr_ref, x_ref, out_ref):
    out_ref[...] = jax.lax.dot_general(
        wr_ref[...], x_ref[...], (((1,), (1,)), ((), ())),
        preferred_element_type=jnp.float32,
        precision=jax.lax.Precision.DEFAULT)


def _router_logits_T(x_flat, W_r):
    N, D = x_flat.shape
    E = W_r.shape[0]
    return pl.pallas_call(
        _router_body,
        grid=(N // N_TOK_BLK,),
        in_specs=[
            pl.BlockSpec((E, D), lambda i: (0, 0)),
            pl.BlockSpec((N_TOK_BLK, D), lambda i: (i, 0)),
        ],
        out_specs=pl.BlockSpec((E, N_TOK_BLK), lambda i: (0, i)),
        out_shape=jax.ShapeDtypeStruct((E, N), jnp.float32),
    )(W_r, x_flat)


def _route_body(logits_ref, sel_ref, fcs_ref, keys_ref):
    E, N = logits_ref.shape
    b = jax.lax.bitcast_convert_type(logits_ref[...], jnp.int32)
    keys = jnp.where(b >= 0, b, b ^ jnp.int32(0x7FFFFFFF))
    keys_ref[...] = keys

    def vstep(i, p):
        bit = jnp.int32(31) - i
        cand = p | jnp.left_shift(jnp.int32(1), bit)
        thresh = cand ^ jnp.int32(SIGN)
        cnt = jnp.sum((keys_ref[...] >= thresh).astype(jnp.int32), axis=1,
                      keepdims=True)
        return jnp.where(cnt >= CAP, cand, p)

    p = jax.lax.fori_loop(0, 32, vstep, jnp.zeros((E, 1), jnp.int32))
    tkey = p ^ jnp.int32(SIGN)
    m_gt = keys_ref[...] > tkey
    m_eq = keys_ref[...] == tkey
    cnt_gt = jnp.sum(m_gt.astype(jnp.int32), axis=1, keepdims=True)
    quota = jnp.int32(CAP) - cnt_gt

    n_iota = jax.lax.broadcasted_iota(jnp.int32, (E, N), 1)
    r_iota = jnp.int32(N - 1) - n_iota

    def istep(i, pr):
        bit = jnp.int32(14) - i
        cand = pr | jnp.left_shift(jnp.int32(1), bit)
        cnt = jnp.sum((m_eq & (r_iota >= cand)).astype(jnp.int32), axis=1,
                      keepdims=True)
        return jnp.where(cnt >= quota, cand, pr)

    pr = jax.lax.fori_loop(0, 15, istep, jnp.zeros((E, 1), jnp.int32))
    idx_thresh = jnp.int32(N - 1) - pr
    sel = m_gt | (m_eq & (n_iota <= idx_thresh) & (quota > 0))
    sel_i = sel.astype(jnp.int32)
    sel_ref[...] = sel_i

    cs = sel_i
    s = 1
    while s < N:
        cs = cs + jnp.pad(cs, ((0, 0), (s, 0)))[:, :N]
        s *= 2
    fcs_ref[...] = cs


def _route(logitsT):
    E, N = logitsT.shape
    return pl.pallas_call(
        _route_body,
        in_specs=[pl.BlockSpec((E, N), lambda: (0, 0))],
        out_specs=[
            pl.BlockSpec((E, N), lambda: (0, 0)),
            pl.BlockSpec((E, N), lambda: (0, 0)),
        ],
        out_shape=[
            jax.ShapeDtypeStruct((E, N), jnp.int32),
            jax.ShapeDtypeStruct((E, N), jnp.int32),
        ],
        scratch_shapes=[pltpu.VMEM((E, N), jnp.int32)],
    )(logitsT)


def _compact(sel, fcs):
    E, N = sel.shape
    mesh = plsc.VectorSubcoreMesh(core_axis_name="c", subcore_axis_name="s")
    epw = E // NW

    @functools.partial(
        pl.kernel, mesh=mesh,
        out_type=jax.ShapeDtypeStruct((E, CAP), jnp.int32),
        scratch_types=[
            pltpu.VMEM((N,), jnp.int32),
            pltpu.VMEM((N,), jnp.int32),
            pltpu.VMEM((CAP + 16,), jnp.int32),
        ],
    )
    def k(sel_hbm, fcs_hbm, idx_hbm, sel_v, fcs_v, out_v):
        wid = lax.axis_index("s") * NC + lax.axis_index("c")
        lane0 = lax.iota(jnp.int32, 16) == 0
        for j in range(epw):
            e = wid * epw + j
            pltpu.sync_copy(sel_hbm.at[e], sel_v)
            pltpu.sync_copy(fcs_hbm.at[e], fcs_v)

            def blk(q, o):
                cs_end = fcs_v[pl.ds(q * 16, 16)][15]

                def hit(o):
                    sv = sel_v[pl.ds(q * 16, 16)]
                    for ll in range(16):
                        def wr(o, ll=ll):
                            prev = out_v[pl.ds(o, 16)]
                            out_v[pl.ds(o, 16)] = jnp.where(
                                lane0, q * 16 + ll, prev)
                            return o + 1

                        o = lax.cond(sv[ll] > 0, wr, lambda a: a, o)
                    return o

                return lax.cond(cs_end > o, hit, lambda a: a, o)

            lax.fori_loop(0, N // 16, blk, jnp.int32(0))
            pltpu.sync_copy(out_v.at[pl.ds(0, CAP)], idx_hbm.at[e])

    return k(sel, fcs)


def _gather_rows(x_flat, flat_idx):
    N, D = x_flat.shape
    M = flat_idx.shape[0]
    mesh = plsc.VectorSubcoreMesh(core_axis_name="c", subcore_axis_name="s")
    rpw = M // NW
    BG = 64

    @functools.partial(
        pl.kernel, mesh=mesh,
        out_type=jax.ShapeDtypeStruct((M, D), jnp.float32),
        scratch_types=[
            pltpu.VMEM((BG,), jnp.int32),
            pltpu.VMEM((BG, D), jnp.float32),
            pltpu.SemaphoreType.DMA,
        ],
    )
    def k(x_hbm, fi_hbm, xe_hbm, idx_v, rows_v, sem):
        wid = lax.axis_index("s") * NC + lax.axis_index("c")
        for b in range(rpw // BG):
            base = wid * rpw + b * BG
            pltpu.sync_copy(fi_hbm.at[pl.ds(base, BG)], idx_v)
            pltpu.async_copy(x_hbm.at[idx_v], rows_v, sem).wait()
            pltpu.sync_copy(rows_v, xe_hbm.at[pl.ds(base, BG)])

    return k(x_flat, flat_idx)


def _ffn_body(xe_ref, w1_ref, b1_ref, w2_ref, b2_ref, ye_ref):
    f = pl.program_id(1)
    xe = xe_ref[0]
    h = jax.lax.dot_general(
        xe, w1_ref[0], (((1,), (1,)), ((), ())),
        preferred_element_type=jnp.float32,
        precision=jax.lax.Precision.DEFAULT) + b1_ref[0]
    h = 0.5 * h * (1.0 + jax.lax.erf(h * 0.7071067811865476))
    contrib = jax.lax.dot_general(
        h, w2_ref[0], (((1,), (1,)), ((), ())),
        preferred_element_type=jnp.float32,
        precision=jax.lax.Precision.DEFAULT)

    @pl.when(f == 0)
    def _():
        ye_ref[0] = contrib + b2_ref[0]

    @pl.when(f != 0)
    def _():
        ye_ref[0] += contrib


def _ffn(xe, W1, b1, W2, b2):
    E, cap, D = xe.shape
    D_ff = W1.shape[1]
    return pl.pallas_call(
        _ffn_body,
        grid=(E, D_ff // FF_BLK),
        in_specs=[
            pl.BlockSpec((1, cap, D), lambda e, f: (e, 0, 0)),
            pl.BlockSpec((1, FF_BLK, D), lambda e, f: (e, f, 0)),
            pl.BlockSpec((1, 1, FF_BLK), lambda e, f: (e, 0, f)),
            pl.BlockSpec((1, D, FF_BLK), lambda e, f: (e, 0, f)),
            pl.BlockSpec((1, 1, D), lambda e, f: (e, 0, 0)),
        ],
        out_specs=pl.BlockSpec((1, cap, D), lambda e, f: (e, 0, 0)),
        out_shape=jax.ShapeDtypeStruct((E, cap, D), jnp.float32),
    )(xe, W1, b1.reshape(E, 1, D_ff), W2, b2.reshape(E, 1, D))


def _combine(ye_flat, flat_idx, fcs1d, N, D, E):
    M = flat_idx.shape[0]
    mesh = plsc.VectorSubcoreMesh(core_axis_name="c", subcore_axis_name="s")
    tpw = N // NW
    CH = 64
    NCH = tpw // CH
    SEG = tpw + 32

    @functools.partial(
        pl.kernel, mesh=mesh,
        out_type=jax.ShapeDtypeStruct((N * D,), jnp.float32),
        scratch_types=[
            pltpu.VMEM((M + 16,), jnp.int32),
            pltpu.VMEM((E * SEG,), jnp.int32),
            pltpu.VMEM((D,), jnp.float32),
            pltpu.VMEM((CH * D,), jnp.float32),
        ],
    )
    def k(ye_hbm, fi_hbm, fcs_hbm, out_hbm, fidx_v, seg_v, row_v, obuf_v):
        wid = lax.axis_index("s") * NC + lax.axis_index("c")
        n0 = wid * tpw
        pltpu.sync_copy(fi_hbm, fidx_v.at[pl.ds(0, M)])

        def pre(e, _):
            def w0(_):
                pltpu.sync_copy(fcs_hbm.at[pl.ds(e * N, tpw + 16)],
                                seg_v.at[pl.ds(e * SEG + 16, tpw + 16)])
                return 0

            def wn(_):
                pltpu.sync_copy(fcs_hbm.at[pl.ds(e * N + n0 - 16, tpw + 16)],
                                seg_v.at[pl.ds(e * SEG, tpw + 16)])
                return 0

            lax.cond(n0 == 0, w0, wn, 0)
            return 0

        lax.fori_loop(0, E, pre, 0)

        def chunk_body(c, _):
            c0 = c * CH

            def z(i, _):
                obuf_v[pl.ds(i * 16, 16)] = jnp.zeros((16,), jnp.float32)
                return 0

            lax.fori_loop(0, CH * D // 16, z, 0)

            def per_e(e, _):
                base = e * SEG + c0
                a_l = seg_v[pl.ds(base, 16)][15]
                a = jnp.where((n0 == 0) & (c == 0), jnp.int32(0), a_l)
                bv = seg_v[pl.ds(base + CH, 16)][15]

                def havework(_):
                    def row(r, _):
                        src = e * CAP + r
                        tok = fidx_v[pl.ds(src, 16)][0] - (n0 + c0)
                        pltpu.sync_copy(ye_hbm.at[src], row_v)
                        ob = tok * D
                        for jj in range(D // 16):
                            obuf_v[pl.ds(ob + jj * 16, 16)] += (
                                row_v[pl.ds(jj * 16, 16)])
                        return 0

                    lax.fori_loop(a, bv, row, 0)
                    return 0

                lax.cond(bv > a, havework, lambda _: 0, 0)
                return 0

            lax.fori_loop(0, E, per_e, 0)
            pltpu.sync_copy(obuf_v, out_hbm.at[pl.ds((n0 + c0) * D, CH * D)])
            return 0

        lax.fori_loop(0, NCH, chunk_body, 0)

    return k(ye_flat, flat_idx, fcs1d)


def kernel(x, W_r, W1, b1, W2, b2):
    Bx, Tx, D = x.shape
    x_flat = x.reshape(-1, D)
    N = x_flat.shape[0]
    E = W_r.shape[0]

    logitsT = _router_logits_T(x_flat, W_r)
    sel, fcs = _route(logitsT)
    indices = _compact(sel, fcs)
    flat_idx = indices.reshape(-1)
    xe_flat = _gather_rows(x_flat, flat_idx)
    ye = _ffn(xe_flat.reshape(E, CAP, D), W1, b1, W2, b2)
    out_flat = _combine(ye.reshape(E * CAP, D), flat_idx, fcs.reshape(-1),
                        N, D, E)
    return out_flat.reshape(Bx, Tx, D)
